# Initial kernel scaffold; baseline (speedup 1.0000x reference)
import jax
import jax.numpy as jnp
from jax.experimental import pallas as pl


def kernel(x_object, x_ssBox, x_place_frame, x_ssCylinder, times_all, actives_all, edge_all, num_pick_nodes, num_place_nodes, W_emb_ssBox, b_emb_ssBox, W_emb_place_frame, b_emb_place_frame, W_emb_object, b_emb_object, W_emb_ssCylinder, b_emb_ssCylinder, W_emb_pick, b_emb_pick, W_emb_place, b_emb_place, Wl, bl, Wr, W_out1, b_out1, W_out2, b_out2):
    raise NotImplementedError("write your pallas kernel here")


# speedup vs baseline: 2.2068x; 2.2068x over previous
import functools
import math

import jax
import jax.numpy as jnp
from jax import lax
from jax.experimental import pallas as pl
from jax.experimental.pallas import tpu as pltpu
from jax.experimental.pallas import tpu_sc as plsc

_ET_PAIRS = [
    (2, 0), (0, 2), (1, 0), (0, 1), (1, 2), (2, 1),
    (4, 5), (5, 4), (2, 2), (0, 0), (1, 1), (3, 3),
    (2, 4), (4, 2), (1, 4), (4, 1), (3, 4), (4, 3),
    (2, 5), (5, 2), (3, 5), (5, 3), (1, 5), (5, 1),
]
_NT = 6
_NET = len(_ET_PAIRS)

_DST_GROUP = [[i for i, (s, d) in enumerate(_ET_PAIRS) if d == t] for t in range(_NT)]


def _sc_mesh():
    return plsc.VectorSubcoreMesh(core_axis_name="c", subcore_axis_name="s")


def _make_prologue(N, E):
    C = 16384
    n_chunks = E // C

    @functools.partial(
        pl.kernel,
        mesh=_sc_mesh(),
        out_type=jax.ShapeDtypeStruct((_NET, N), jnp.float32),
        compiler_params=pltpu.CompilerParams(needs_layout_passes=False),
        scratch_types=[
            pltpu.VMEM((N,), jnp.float32),
            pltpu.VMEM((C,), jnp.int32),
        ],
    )
    def prologue(dst_hbm, cnt_hbm, cnt_v, di_v):
        wid = lax.axis_index("s") * 2 + lax.axis_index("c")

        @pl.when(wid < _NET)
        def _():
            zeros16 = jnp.zeros((16,), jnp.float32)
            ones16 = jnp.ones((16,), jnp.float32)

            def zero_body(k, _):
                cnt_v[pl.ds(k * 16, 16)] = zeros16
                return 0

            lax.fori_loop(0, N // 16, zero_body, 0)

            def chunk_body(c, _):
                pltpu.sync_copy(dst_hbm.at[wid, pl.ds(c * C, C)], di_v)

                def step(k, _):
                    di = di_v[pl.ds(k * 16, 16)]
                    plsc.addupdate_scatter(cnt_v, [di], ones16)
                    return 0

                lax.fori_loop(0, C // 16, step, 0)
                return 0

            lax.fori_loop(0, n_chunks, chunk_body, 0)
            pltpu.sync_copy(cnt_v, cnt_hbm.at[wid])

    return prologue


def _make_sc_layer(N, E):
    C = 16384
    n_chunks = E // C
    U = 4

    @functools.partial(
        pl.kernel,
        mesh=_sc_mesh(),
        out_type=jax.ShapeDtypeStruct((_NET, 64, N), jnp.float32),
        compiler_params=pltpu.CompilerParams(needs_layout_passes=False),
        scratch_types=[
            pltpu.VMEM((2, N), jnp.float32),
            pltpu.VMEM((2, N), jnp.float32),
            pltpu.VMEM((C,), jnp.int32),
            pltpu.VMEM((C,), jnp.int32),
        ],
    )
    def sc_layer(xs_hbm, src_hbm, dst_hbm, out_hbm, acc_v, y_v, si_v, di_v):
        wid = lax.axis_index("s") * 2 + lax.axis_index("c")
        d0 = wid * 2
        d_idx = [jnp.full((16,), d, jnp.int32) for d in range(2)]
        zeros16 = jnp.zeros((16,), jnp.float32)

        for i in range(_NET):
            s_type = _ET_PAIRS[i][0]
            pltpu.sync_copy(xs_hbm.at[s_type, pl.ds(d0, 2)], y_v)

            def zero_body(k, _):
                b = k * 16
                for d in range(2):
                    acc_v[d, pl.ds(b, 16)] = zeros16
                return 0

            lax.fori_loop(0, N // 16, zero_body, 0)

            def chunk_body(c, _):
                pltpu.sync_copy(src_hbm.at[i, pl.ds(c * C, C)], si_v)
                pltpu.sync_copy(dst_hbm.at[i, pl.ds(c * C, C)], di_v)

                def step(k, _):
                    for u in range(U):
                        b = (k * U + u) * 16
                        si = si_v[pl.ds(b, 16)]
                        di = di_v[pl.ds(b, 16)]
                        for d in range(2):
                            v = plsc.load_gather(y_v, [d_idx[d], si])
                            plsc.addupdate_scatter(acc_v, [d_idx[d], di], v)
                    return 0

                lax.fori_loop(0, C // (16 * U), step, 0)
                return 0

            lax.fori_loop(0, n_chunks, chunk_body, 0)
            pltpu.sync_copy(acc_v, out_hbm.at[i, pl.ds(d0, 2)])

    return sc_layer


import numpy as _np

_2PI = 2.0 * math.pi
_P1 = float(_np.float32(6.28125))
_P2 = float(_np.float32(_2PI - _P1))
_P3 = float(_np.float32(_2PI - _P1 - float(_np.float32(_2PI - _P1))))


def _dot(a, b):
    return jax.lax.dot_general(
        a.astype(jnp.bfloat16), b.astype(jnp.bfloat16),
        (((1,), (0,)), ((), ())),
        preferred_element_type=jnp.float32)


def _dot_hi(a, b):
    return jax.lax.dot_general(
        a, b, (((1,), (0,)), ((), ())),
        precision=jax.lax.Precision.HIGHEST,
        preferred_element_type=jnp.float32)


def _sincos(x):
    k = jnp.floor(x * (1.0 / _2PI) + 0.5)
    r = ((x - k * _P1) - k * _P2) - k * _P3
    return jnp.sin(r), jnp.cos(r)


def _emb_body(fbox, fpf, fobj, fcyl, times, wemb, bemb, out):
    feats = {0: fbox, 1: fpf, 2: fobj, 3: fcyl}
    bn = times.shape[1]
    for t in range(_NT):
        tt = times[t:t + 1, :].astype(jnp.float32)
        s1, c1 = _sincos(tt)
        s2, c2 = _sincos(tt * 0.01)
        pe = [s1, c1, s2, c2]
        if t <= 2:
            rows = [feats[t][:, :]] + pe
        elif t == 3:
            rows = [feats[3][:, :]] + pe + [jnp.zeros((1, bn), jnp.float32)]
        else:
            rows = pe + [jnp.zeros((4, bn), jnp.float32)]
        inp = jnp.concatenate(rows, axis=0)
        out[t] = _dot(wemb[t], inp) + bemb[t][:, None]


def _make_emb(N):
    BN = 2048
    return pl.pallas_call(
        _emb_body,
        grid=(N // BN,),
        in_specs=[
            pl.BlockSpec((4, BN), lambda nb: (0, nb)),
            pl.BlockSpec((4, BN), lambda nb: (0, nb)),
            pl.BlockSpec((4, BN), lambda nb: (0, nb)),
            pl.BlockSpec((3, BN), lambda nb: (0, nb)),
            pl.BlockSpec((_NT, BN), lambda nb: (0, nb)),
            pl.BlockSpec((_NT, 64, 8), lambda nb: (0, 0, 0)),
            pl.BlockSpec((_NT, 64), lambda nb: (0, 0)),
        ],
        out_specs=pl.BlockSpec((_NT, 64, BN), lambda nb: (0, 0, nb)),
        out_shape=jax.ShapeDtypeStruct((_NT, 64, N), jnp.float32),
    )


def _layer_body(s, c, xs, wl, wr, bias, out):
    for t in range(_NT):
        o = None
        for i in _DST_GROUP[t]:
            cc = jnp.maximum(c[i:i + 1, :], 1.0)
            agg = s[i] / cc
            m = _dot(wl[i], agg) + bias[i][:, None]
            m = m + _dot(wr[i], xs[t])
            o = m if o is None else o + m
        out[t] = jnp.maximum(o, 0.0)


def _make_layer_tc(N):
    BN = 1024
    return pl.pallas_call(
        _layer_body,
        grid=(N // BN,),
        in_specs=[
            pl.BlockSpec((_NET, 64, BN), lambda nb: (0, 0, nb)),
            pl.BlockSpec((_NET, BN), lambda nb: (0, nb)),
            pl.BlockSpec((_NT, 64, BN), lambda nb: (0, 0, nb)),
            pl.BlockSpec((_NET, 64, 64), lambda nb: (0, 0, 0)),
            pl.BlockSpec((_NET, 64, 64), lambda nb: (0, 0, 0)),
            pl.BlockSpec((_NET, 64), lambda nb: (0, 0)),
        ],
        out_specs=pl.BlockSpec((_NT, 64, BN), lambda nb: (0, 0, nb)),
        out_shape=jax.ShapeDtypeStruct((_NT, 64, N), jnp.float32),
    )


def _head_body(x, w1, b1, w2, b2, out):
    h = jnp.maximum(x[0], 0.0)
    o1 = _dot(w1[...], h) + b1[0][:, None]
    o1 = jnp.maximum(o1, 0.0)
    o2 = _dot(w2[...], o1) + b2[0][:, None]
    out[0] = o2


def _make_head(N, HQ):
    BN = 2048
    return pl.pallas_call(
        _head_body,
        grid=(2, N // BN),
        in_specs=[
            pl.BlockSpec((1, 64, BN), lambda j, nb: (4 + j, 0, nb)),
            pl.BlockSpec((HQ, 64), lambda j, nb: (0, 0)),
            pl.BlockSpec((1, HQ), lambda j, nb: (0, 0)),
            pl.BlockSpec((1, HQ), lambda j, nb: (0, 0)),
            pl.BlockSpec((1, 1), lambda j, nb: (0, 0)),
        ],
        out_specs=pl.BlockSpec((1, 1, BN), lambda j, nb: (j, 0, nb)),
        out_shape=jax.ShapeDtypeStruct((2, 1, N), jnp.float32),
    )


def kernel(x_object, x_ssBox, x_place_frame, x_ssCylinder, times_all,
           actives_all, edge_all, num_pick_nodes, num_place_nodes,
           W_emb_ssBox, b_emb_ssBox, W_emb_place_frame, b_emb_place_frame,
           W_emb_object, b_emb_object, W_emb_ssCylinder, b_emb_ssCylinder,
           W_emb_pick, b_emb_pick, W_emb_place, b_emb_place,
           Wl, bl, Wr, W_out1, b_out1, W_out2, b_out2):
    N = x_object.shape[0]
    E = edge_all.shape[2]
    L = Wl.shape[0]

    times = jnp.minimum(times_all, 2 * num_pick_nodes - 1).astype(jnp.int32)
    src24 = edge_all[:, 0, :].astype(jnp.int32)
    dst24 = edge_all[:, 1, :].astype(jnp.int32)

    fbox = x_ssBox.T
    fpf = x_place_frame.T
    fobj = x_object.T
    fcyl = x_ssCylinder.T

    pad_c = lambda w, k: jnp.pad(w, ((0, 0), (0, k)))
    wemb = jnp.stack([
        W_emb_ssBox, W_emb_place_frame, W_emb_object,
        pad_c(W_emb_ssCylinder, 1), pad_c(W_emb_pick, 4), pad_c(W_emb_place, 4),
    ])
    bemb = jnp.stack([
        b_emb_ssBox, b_emb_place_frame, b_emb_object,
        b_emb_ssCylinder, b_emb_pick, b_emb_place,
    ])

    cnt = _make_prologue(N, E)(dst24)
    xs = _make_emb(N)(fbox, fpf, fobj, fcyl, times, wemb, bemb)

    layer_tc = _make_layer_tc(N)
    sc_layer = _make_sc_layer(N, E)
    for l in range(L):
        s = sc_layer(xs, src24, dst24)
        xs = layer_tc(s, cnt, xs, Wl[l], Wr[l], bl[l])

    hq = W_out1.shape[0]
    out = _make_head(N, hq)(xs, W_out1, b_out1[None, :], W_out2,
                            b_out2[None, :])
    return out.reshape(2 * N, 1)


# speedup vs baseline: 4.8856x; 2.2139x over previous
import functools
import math

import jax
import jax.numpy as jnp
from jax import lax
from jax.experimental import pallas as pl
from jax.experimental.pallas import tpu as pltpu
from jax.experimental.pallas import tpu_sc as plsc

_ET_PAIRS = [
    (2, 0), (0, 2), (1, 0), (0, 1), (1, 2), (2, 1),
    (4, 5), (5, 4), (2, 2), (0, 0), (1, 1), (3, 3),
    (2, 4), (4, 2), (1, 4), (4, 1), (3, 4), (4, 3),
    (2, 5), (5, 2), (3, 5), (5, 3), (1, 5), (5, 1),
]
_NT = 6
_NET = len(_ET_PAIRS)

_DST_GROUP = [[i for i, (s, d) in enumerate(_ET_PAIRS) if d == t] for t in range(_NT)]


def _sc_mesh():
    return plsc.VectorSubcoreMesh(core_axis_name="c", subcore_axis_name="s")


def _make_prologue(N, E):
    C = 16384
    n_chunks = E // C

    @functools.partial(
        pl.kernel,
        mesh=_sc_mesh(),
        out_type=jax.ShapeDtypeStruct((_NET, N), jnp.float32),
        compiler_params=pltpu.CompilerParams(needs_layout_passes=False),
        scratch_types=[
            pltpu.VMEM((N,), jnp.float32),
            pltpu.VMEM((C,), jnp.int32),
        ],
    )
    def prologue(dst_hbm, cnt_hbm, cnt_v, di_v):
        wid = lax.axis_index("s") * 2 + lax.axis_index("c")

        @pl.when(wid < _NET)
        def _():
            zeros16 = jnp.zeros((16,), jnp.float32)
            ones16 = jnp.ones((16,), jnp.float32)

            def zero_body(k, _):
                cnt_v[pl.ds(k * 16, 16)] = zeros16
                return 0

            lax.fori_loop(0, N // 16, zero_body, 0)

            def chunk_body(c, _):
                pltpu.sync_copy(dst_hbm.at[wid, pl.ds(c * C, C)], di_v)

                def step(k, _):
                    di = di_v[pl.ds(k * 16, 16)]
                    plsc.addupdate_scatter(cnt_v, [di], ones16)
                    return 0

                lax.fori_loop(0, C // 16, step, 0)
                return 0

            lax.fori_loop(0, n_chunks, chunk_body, 0)
            pltpu.sync_copy(cnt_v, cnt_hbm.at[wid])

    return prologue


def _make_sc_layer(N, E):
    C = 16384
    n_chunks = E // C
    U = 8

    @functools.partial(
        pl.kernel,
        mesh=_sc_mesh(),
        out_type=jax.ShapeDtypeStruct((_NET, 64, N), jnp.float32),
        compiler_params=pltpu.CompilerParams(needs_layout_passes=False),
        scratch_types=[
            pltpu.VMEM((2, N), jnp.float32),
            pltpu.VMEM((2, N), jnp.float32),
            pltpu.VMEM((C,), jnp.int32),
            pltpu.VMEM((C,), jnp.int32),
        ],
    )
    def sc_layer(xs_hbm, src_hbm, dst_hbm, out_hbm, acc_v, y_v, si_v, di_v):
        wid = lax.axis_index("s") * 2 + lax.axis_index("c")
        d0 = wid * 2
        d_idx = [jnp.full((16,), d, jnp.int32) for d in range(2)]
        zeros16 = jnp.zeros((16,), jnp.float32)

        for i in range(_NET):
            s_type = _ET_PAIRS[i][0]
            pltpu.sync_copy(xs_hbm.at[s_type, pl.ds(d0, 2)], y_v)

            @plsc.parallel_loop(0, N // 16, unroll=8)
            def _(k):
                b = k * 16
                for d in range(2):
                    acc_v[d, pl.ds(b, 16)] = zeros16

            def chunk_body(c, _):
                pltpu.sync_copy(src_hbm.at[i, pl.ds(c * C, C)], si_v)
                pltpu.sync_copy(dst_hbm.at[i, pl.ds(c * C, C)], di_v)

                @plsc.parallel_loop(0, C // 16, unroll=U)
                def _(k):
                    b = k * 16
                    si = si_v[pl.ds(b, 16)]
                    di = di_v[pl.ds(b, 16)]
                    for d in range(2):
                        v = plsc.load_gather(y_v, [d_idx[d], si])
                        plsc.addupdate_scatter(acc_v, [d_idx[d], di], v)
                return 0

            lax.fori_loop(0, n_chunks, chunk_body, 0)
            pltpu.sync_copy(acc_v, out_hbm.at[i, pl.ds(d0, 2)])

    return sc_layer


import numpy as _np

_2PI = 2.0 * math.pi
_P1 = float(_np.float32(6.28125))
_P2 = float(_np.float32(_2PI - _P1))
_P3 = float(_np.float32(_2PI - _P1 - float(_np.float32(_2PI - _P1))))


def _dot(a, b):
    return jax.lax.dot_general(
        a.astype(jnp.bfloat16), b.astype(jnp.bfloat16),
        (((1,), (0,)), ((), ())),
        preferred_element_type=jnp.float32)


def _dot_hi(a, b):
    return jax.lax.dot_general(
        a, b, (((1,), (0,)), ((), ())),
        precision=jax.lax.Precision.HIGHEST,
        preferred_element_type=jnp.float32)


def _sincos(x):
    k = jnp.floor(x * (1.0 / _2PI) + 0.5)
    r = ((x - k * _P1) - k * _P2) - k * _P3
    return jnp.sin(r), jnp.cos(r)


def _emb_body(fbox, fpf, fobj, fcyl, times, wemb, bemb, out):
    feats = {0: fbox, 1: fpf, 2: fobj, 3: fcyl}
    bn = times.shape[1]
    for t in range(_NT):
        tt = times[t:t + 1, :].astype(jnp.float32)
        s1, c1 = _sincos(tt)
        s2, c2 = _sincos(tt * 0.01)
        pe = [s1, c1, s2, c2]
        if t <= 2:
            rows = [feats[t][:, :]] + pe
        elif t == 3:
            rows = [feats[3][:, :]] + pe + [jnp.zeros((1, bn), jnp.float32)]
        else:
            rows = pe + [jnp.zeros((4, bn), jnp.float32)]
        inp = jnp.concatenate(rows, axis=0)
        out[t] = _dot(wemb[t], inp) + bemb[t][:, None]


def _make_emb(N):
    BN = 2048
    return pl.pallas_call(
        _emb_body,
        grid=(N // BN,),
        in_specs=[
            pl.BlockSpec((4, BN), lambda nb: (0, nb)),
            pl.BlockSpec((4, BN), lambda nb: (0, nb)),
            pl.BlockSpec((4, BN), lambda nb: (0, nb)),
            pl.BlockSpec((3, BN), lambda nb: (0, nb)),
            pl.BlockSpec((_NT, BN), lambda nb: (0, nb)),
            pl.BlockSpec((_NT, 64, 8), lambda nb: (0, 0, 0)),
            pl.BlockSpec((_NT, 64), lambda nb: (0, 0)),
        ],
        out_specs=pl.BlockSpec((_NT, 64, BN), lambda nb: (0, 0, nb)),
        out_shape=jax.ShapeDtypeStruct((_NT, 64, N), jnp.float32),
    )


def _layer_body(s, c, xs, wl, wr, bias, out):
    for t in range(_NT):
        o = None
        for i in _DST_GROUP[t]:
            cc = jnp.maximum(c[i:i + 1, :], 1.0)
            agg = s[i] / cc
            m = _dot(wl[i], agg) + bias[i][:, None]
            m = m + _dot(wr[i], xs[t])
            o = m if o is None else o + m
        out[t] = jnp.maximum(o, 0.0)


def _make_layer_tc(N):
    BN = 1024
    return pl.pallas_call(
        _layer_body,
        grid=(N // BN,),
        in_specs=[
            pl.BlockSpec((_NET, 64, BN), lambda nb: (0, 0, nb)),
            pl.BlockSpec((_NET, BN), lambda nb: (0, nb)),
            pl.BlockSpec((_NT, 64, BN), lambda nb: (0, 0, nb)),
            pl.BlockSpec((_NET, 64, 64), lambda nb: (0, 0, 0)),
            pl.BlockSpec((_NET, 64, 64), lambda nb: (0, 0, 0)),
            pl.BlockSpec((_NET, 64), lambda nb: (0, 0)),
        ],
        out_specs=pl.BlockSpec((_NT, 64, BN), lambda nb: (0, 0, nb)),
        out_shape=jax.ShapeDtypeStruct((_NT, 64, N), jnp.float32),
    )


def _head_body(x, w1, b1, w2, b2, out):
    h = jnp.maximum(x[0], 0.0)
    o1 = _dot(w1[...], h) + b1[0][:, None]
    o1 = jnp.maximum(o1, 0.0)
    o2 = _dot(w2[...], o1) + b2[0][:, None]
    out[0] = o2


def _make_head(N, HQ):
    BN = 2048
    return pl.pallas_call(
        _head_body,
        grid=(2, N // BN),
        in_specs=[
            pl.BlockSpec((1, 64, BN), lambda j, nb: (4 + j, 0, nb)),
            pl.BlockSpec((HQ, 64), lambda j, nb: (0, 0)),
            pl.BlockSpec((1, HQ), lambda j, nb: (0, 0)),
            pl.BlockSpec((1, HQ), lambda j, nb: (0, 0)),
            pl.BlockSpec((1, 1), lambda j, nb: (0, 0)),
        ],
        out_specs=pl.BlockSpec((1, 1, BN), lambda j, nb: (j, 0, nb)),
        out_shape=jax.ShapeDtypeStruct((2, 1, N), jnp.float32),
    )


def kernel(x_object, x_ssBox, x_place_frame, x_ssCylinder, times_all,
           actives_all, edge_all, num_pick_nodes, num_place_nodes,
           W_emb_ssBox, b_emb_ssBox, W_emb_place_frame, b_emb_place_frame,
           W_emb_object, b_emb_object, W_emb_ssCylinder, b_emb_ssCylinder,
           W_emb_pick, b_emb_pick, W_emb_place, b_emb_place,
           Wl, bl, Wr, W_out1, b_out1, W_out2, b_out2):
    N = x_object.shape[0]
    E = edge_all.shape[2]
    L = Wl.shape[0]

    times = jnp.minimum(times_all, 2 * num_pick_nodes - 1).astype(jnp.int32)
    src24 = edge_all[:, 0, :].astype(jnp.int32)
    dst24 = edge_all[:, 1, :].astype(jnp.int32)

    fbox = x_ssBox.T
    fpf = x_place_frame.T
    fobj = x_object.T
    fcyl = x_ssCylinder.T

    pad_c = lambda w, k: jnp.pad(w, ((0, 0), (0, k)))
    wemb = jnp.stack([
        W_emb_ssBox, W_emb_place_frame, W_emb_object,
        pad_c(W_emb_ssCylinder, 1), pad_c(W_emb_pick, 4), pad_c(W_emb_place, 4),
    ])
    bemb = jnp.stack([
        b_emb_ssBox, b_emb_place_frame, b_emb_object,
        b_emb_ssCylinder, b_emb_pick, b_emb_place,
    ])

    cnt = _make_prologue(N, E)(dst24)
    xs = _make_emb(N)(fbox, fpf, fobj, fcyl, times, wemb, bemb)

    layer_tc = _make_layer_tc(N)
    sc_layer = _make_sc_layer(N, E)
    for l in range(L):
        s = sc_layer(xs, src24, dst24)
        xs = layer_tc(s, cnt, xs, Wl[l], Wr[l], bl[l])

    hq = W_out1.shape[0]
    out = _make_head(N, hq)(xs, W_out1, b_out1[None, :], W_out2,
                            b_out2[None, :])
    return out.reshape(2 * N, 1)


# speedup vs baseline: 6.2084x; 1.2707x over previous
import functools
import math

import jax
import jax.numpy as jnp
from jax import lax
from jax.experimental import pallas as pl
from jax.experimental.pallas import tpu as pltpu
from jax.experimental.pallas import tpu_sc as plsc

_ET_PAIRS = [
    (2, 0), (0, 2), (1, 0), (0, 1), (1, 2), (2, 1),
    (4, 5), (5, 4), (2, 2), (0, 0), (1, 1), (3, 3),
    (2, 4), (4, 2), (1, 4), (4, 1), (3, 4), (4, 3),
    (2, 5), (5, 2), (3, 5), (5, 3), (1, 5), (5, 1),
]
_NT = 6
_NET = len(_ET_PAIRS)

_DST_GROUP = [[i for i, (s, d) in enumerate(_ET_PAIRS) if d == t] for t in range(_NT)]


def _sc_mesh():
    return plsc.VectorSubcoreMesh(core_axis_name="c", subcore_axis_name="s")


def _make_prologue(N, E):
    C = 16384
    n_chunks = E // C

    @functools.partial(
        pl.kernel,
        mesh=_sc_mesh(),
        out_type=jax.ShapeDtypeStruct((_NET, N), jnp.float32),
        compiler_params=pltpu.CompilerParams(needs_layout_passes=False),
        scratch_types=[
            pltpu.VMEM((N,), jnp.float32),
            pltpu.VMEM((C,), jnp.int32),
        ],
    )
    def prologue(dst_hbm, cnt_hbm, cnt_v, di_v):
        wid = lax.axis_index("s") * 2 + lax.axis_index("c")

        @pl.when(wid < _NET)
        def _():
            zeros16 = jnp.zeros((16,), jnp.float32)
            ones16 = jnp.ones((16,), jnp.float32)

            @plsc.parallel_loop(0, N // 16, unroll=8)
            def _(k):
                cnt_v[pl.ds(k * 16, 16)] = zeros16

            def chunk_body(c, _):
                pltpu.sync_copy(dst_hbm.at[wid, pl.ds(c * C, C)], di_v)

                @plsc.parallel_loop(0, C // 16, unroll=8)
                def _(k):
                    di = di_v[pl.ds(k * 16, 16)]
                    plsc.addupdate_scatter(cnt_v, [di], ones16)

                return 0

            lax.fori_loop(0, n_chunks, chunk_body, 0)
            pltpu.sync_copy(cnt_v, cnt_hbm.at[wid])

    return prologue


def _make_sc_layer(N, E):
    C = 8192
    n_chunks = E // C
    n_pairs = n_chunks // 2
    U = 8

    @functools.partial(
        pl.kernel,
        mesh=_sc_mesh(),
        out_type=jax.ShapeDtypeStruct((_NET, 64, N), jnp.float32),
        compiler_params=pltpu.CompilerParams(needs_layout_passes=False),
        scratch_types=[
            pltpu.VMEM((2, N), jnp.float32),
            pltpu.VMEM((2, N), jnp.float32),
            pltpu.VMEM((C,), jnp.int32),
            pltpu.VMEM((C,), jnp.int32),
            pltpu.VMEM((C,), jnp.int32),
            pltpu.VMEM((C,), jnp.int32),
            pltpu.SemaphoreType.DMA,
            pltpu.SemaphoreType.DMA,
        ],
    )
    def sc_layer(xs_hbm, src_hbm, dst_hbm, out_hbm, acc_v, y_v,
                 si_a, di_a, si_b, di_b, sem_a, sem_b):
        wid = lax.axis_index("s") * 2 + lax.axis_index("c")
        d0 = wid * 2
        d_idx = [jnp.full((16,), d, jnp.int32) for d in range(2)]
        zeros16 = jnp.zeros((16,), jnp.float32)

        def issue(i, c, sv, dv, sem):
            pltpu.async_copy(src_hbm.at[i, pl.ds(c * C, C)], sv, sem)
            pltpu.async_copy(dst_hbm.at[i, pl.ds(c * C, C)], dv, sem)

        def drain(i, sv, dv, sem):
            pltpu.make_async_copy(src_hbm.at[i, pl.ds(0, C)], sv, sem).wait()
            pltpu.make_async_copy(dst_hbm.at[i, pl.ds(0, C)], dv, sem).wait()

        def process(sv, dv):
            @plsc.parallel_loop(0, C // 16, unroll=U)
            def _(k):
                b = k * 16
                si = sv[pl.ds(b, 16)]
                di = dv[pl.ds(b, 16)]
                for d in range(2):
                    v = plsc.load_gather(y_v, [d_idx[d], si])
                    plsc.addupdate_scatter(acc_v, [d_idx[d], di], v)

        for i in range(_NET):
            s_type = _ET_PAIRS[i][0]
            pltpu.sync_copy(xs_hbm.at[s_type, pl.ds(d0, 2)], y_v)

            @plsc.parallel_loop(0, N // 16, unroll=8)
            def _(k):
                b = k * 16
                for d in range(2):
                    acc_v[d, pl.ds(b, 16)] = zeros16

            issue(i, 0, si_a, di_a, sem_a)

            def pair_body(g, _):
                issue(i, 2 * g + 1, si_b, di_b, sem_b)
                drain(i, si_a, di_a, sem_a)
                process(si_a, di_a)

                @pl.when(g < n_pairs - 1)
                def _():
                    issue(i, 2 * g + 2, si_a, di_a, sem_a)

                drain(i, si_b, di_b, sem_b)
                process(si_b, di_b)
                return 0

            lax.fori_loop(0, n_pairs, pair_body, 0)
            pltpu.sync_copy(acc_v, out_hbm.at[i, pl.ds(d0, 2)])

    return sc_layer


import numpy as _np

_2PI = 2.0 * math.pi
_P1 = float(_np.float32(6.28125))
_P2 = float(_np.float32(_2PI - _P1))
_P3 = float(_np.float32(_2PI - _P1 - float(_np.float32(_2PI - _P1))))


def _dot(a, b):
    return jax.lax.dot_general(
        a.astype(jnp.bfloat16), b.astype(jnp.bfloat16),
        (((1,), (0,)), ((), ())),
        preferred_element_type=jnp.float32)


def _dot_hi(a, b):
    return jax.lax.dot_general(
        a, b, (((1,), (0,)), ((), ())),
        precision=jax.lax.Precision.HIGHEST,
        preferred_element_type=jnp.float32)


def _sincos(x):
    k = jnp.floor(x * (1.0 / _2PI) + 0.5)
    r = ((x - k * _P1) - k * _P2) - k * _P3
    return jnp.sin(r), jnp.cos(r)


def _emb_body(fbox, fpf, fobj, fcyl, times, wemb, bemb, out):
    feats = {0: fbox, 1: fpf, 2: fobj, 3: fcyl}
    bn = times.shape[1]
    for t in range(_NT):
        tt = times[t:t + 1, :].astype(jnp.float32)
        s1, c1 = _sincos(tt)
        s2, c2 = _sincos(tt * 0.01)
        pe = [s1, c1, s2, c2]
        if t <= 2:
            rows = [feats[t][:, :]] + pe
        elif t == 3:
            rows = [feats[3][:, :]] + pe + [jnp.zeros((1, bn), jnp.float32)]
        else:
            rows = pe + [jnp.zeros((4, bn), jnp.float32)]
        inp = jnp.concatenate(rows, axis=0)
        out[t] = _dot(wemb[t], inp) + bemb[t][:, None]


def _make_emb(N):
    BN = 2048
    return pl.pallas_call(
        _emb_body,
        grid=(N // BN,),
        in_specs=[
            pl.BlockSpec((4, BN), lambda nb: (0, nb)),
            pl.BlockSpec((4, BN), lambda nb: (0, nb)),
            pl.BlockSpec((4, BN), lambda nb: (0, nb)),
            pl.BlockSpec((3, BN), lambda nb: (0, nb)),
            pl.BlockSpec((_NT, BN), lambda nb: (0, nb)),
            pl.BlockSpec((_NT, 64, 8), lambda nb: (0, 0, 0)),
            pl.BlockSpec((_NT, 64), lambda nb: (0, 0)),
        ],
        out_specs=pl.BlockSpec((_NT, 64, BN), lambda nb: (0, 0, nb)),
        out_shape=jax.ShapeDtypeStruct((_NT, 64, N), jnp.float32),
    )


def _layer_body(s, c, xs, wl, wr, bias, out):
    for t in range(_NT):
        o = None
        for i in _DST_GROUP[t]:
            cc = jnp.maximum(c[i:i + 1, :], 1.0)
            agg = s[i] / cc
            m = _dot(wl[i], agg) + bias[i][:, None]
            m = m + _dot(wr[i], xs[t])
            o = m if o is None else o + m
        out[t] = jnp.maximum(o, 0.0)


def _make_layer_tc(N):
    BN = 1024
    return pl.pallas_call(
        _layer_body,
        grid=(N // BN,),
        in_specs=[
            pl.BlockSpec((_NET, 64, BN), lambda nb: (0, 0, nb)),
            pl.BlockSpec((_NET, BN), lambda nb: (0, nb)),
            pl.BlockSpec((_NT, 64, BN), lambda nb: (0, 0, nb)),
            pl.BlockSpec((_NET, 64, 64), lambda nb: (0, 0, 0)),
            pl.BlockSpec((_NET, 64, 64), lambda nb: (0, 0, 0)),
            pl.BlockSpec((_NET, 64), lambda nb: (0, 0)),
        ],
        out_specs=pl.BlockSpec((_NT, 64, BN), lambda nb: (0, 0, nb)),
        out_shape=jax.ShapeDtypeStruct((_NT, 64, N), jnp.float32),
    )


def _head_body(x, w1, b1, w2, b2, out):
    h = jnp.maximum(x[0], 0.0)
    o1 = _dot(w1[...], h) + b1[0][:, None]
    o1 = jnp.maximum(o1, 0.0)
    o2 = _dot(w2[...], o1) + b2[0][:, None]
    out[0] = o2


def _make_head(N, HQ):
    BN = 2048
    return pl.pallas_call(
        _head_body,
        grid=(2, N // BN),
        in_specs=[
            pl.BlockSpec((1, 64, BN), lambda j, nb: (4 + j, 0, nb)),
            pl.BlockSpec((HQ, 64), lambda j, nb: (0, 0)),
            pl.BlockSpec((1, HQ), lambda j, nb: (0, 0)),
            pl.BlockSpec((1, HQ), lambda j, nb: (0, 0)),
            pl.BlockSpec((1, 1), lambda j, nb: (0, 0)),
        ],
        out_specs=pl.BlockSpec((1, 1, BN), lambda j, nb: (j, 0, nb)),
        out_shape=jax.ShapeDtypeStruct((2, 1, N), jnp.float32),
    )


def kernel(x_object, x_ssBox, x_place_frame, x_ssCylinder, times_all,
           actives_all, edge_all, num_pick_nodes, num_place_nodes,
           W_emb_ssBox, b_emb_ssBox, W_emb_place_frame, b_emb_place_frame,
           W_emb_object, b_emb_object, W_emb_ssCylinder, b_emb_ssCylinder,
           W_emb_pick, b_emb_pick, W_emb_place, b_emb_place,
           Wl, bl, Wr, W_out1, b_out1, W_out2, b_out2):
    N = x_object.shape[0]
    E = edge_all.shape[2]
    L = Wl.shape[0]

    times = jnp.minimum(times_all, 2 * num_pick_nodes - 1).astype(jnp.int32)
    src24 = edge_all[:, 0, :].astype(jnp.int32)
    dst24 = edge_all[:, 1, :].astype(jnp.int32)

    fbox = x_ssBox.T
    fpf = x_place_frame.T
    fobj = x_object.T
    fcyl = x_ssCylinder.T

    pad_c = lambda w, k: jnp.pad(w, ((0, 0), (0, k)))
    wemb = jnp.stack([
        W_emb_ssBox, W_emb_place_frame, W_emb_object,
        pad_c(W_emb_ssCylinder, 1), pad_c(W_emb_pick, 4), pad_c(W_emb_place, 4),
    ])
    bemb = jnp.stack([
        b_emb_ssBox, b_emb_place_frame, b_emb_object,
        b_emb_ssCylinder, b_emb_pick, b_emb_place,
    ])

    cnt = _make_prologue(N, E)(dst24)
    xs = _make_emb(N)(fbox, fpf, fobj, fcyl, times, wemb, bemb)

    layer_tc = _make_layer_tc(N)
    sc_layer = _make_sc_layer(N, E)
    for l in range(L):
        s = sc_layer(xs, src24, dst24)
        xs = layer_tc(s, cnt, xs, Wl[l], Wr[l], bl[l])

    hq = W_out1.shape[0]
    out = _make_head(N, hq)(xs, W_out1, b_out1[None, :], W_out2,
                            b_out2[None, :])
    return out.reshape(2 * N, 1)


# speedup vs baseline: 6.7706x; 1.0906x over previous
import functools
import math

import jax
import jax.numpy as jnp
from jax import lax
from jax.experimental import pallas as pl
from jax.experimental.pallas import tpu as pltpu
from jax.experimental.pallas import tpu_sc as plsc

_ET_PAIRS = [
    (2, 0), (0, 2), (1, 0), (0, 1), (1, 2), (2, 1),
    (4, 5), (5, 4), (2, 2), (0, 0), (1, 1), (3, 3),
    (2, 4), (4, 2), (1, 4), (4, 1), (3, 4), (4, 3),
    (2, 5), (5, 2), (3, 5), (5, 3), (1, 5), (5, 1),
]
_NT = 6
_NET = len(_ET_PAIRS)

_DST_GROUP = [[i for i, (s, d) in enumerate(_ET_PAIRS) if d == t] for t in range(_NT)]


def _sc_mesh():
    return plsc.VectorSubcoreMesh(core_axis_name="c", subcore_axis_name="s")


def _make_prologue(N, E):
    C = 16384
    n_chunks = E // C

    @functools.partial(
        pl.kernel,
        mesh=_sc_mesh(),
        out_type=jax.ShapeDtypeStruct((_NET, N), jnp.float32),
        compiler_params=pltpu.CompilerParams(needs_layout_passes=False),
        scratch_types=[
            pltpu.VMEM((N,), jnp.float32),
            pltpu.VMEM((C,), jnp.int32),
        ],
    )
    def prologue(dst_hbm, cnt_hbm, cnt_v, di_v):
        wid = lax.axis_index("s") * 2 + lax.axis_index("c")

        @pl.when(wid < _NET)
        def _():
            zeros16 = jnp.zeros((16,), jnp.float32)
            ones16 = jnp.ones((16,), jnp.float32)

            @plsc.parallel_loop(0, N // 16, unroll=8)
            def _(k):
                cnt_v[pl.ds(k * 16, 16)] = zeros16

            def chunk_body(c, _):
                pltpu.sync_copy(dst_hbm.at[wid, pl.ds(c * C, C)], di_v)

                @plsc.parallel_loop(0, C // 16, unroll=8)
                def _(k):
                    di = di_v[pl.ds(k * 16, 16)]
                    plsc.addupdate_scatter(cnt_v, [di], ones16)

                return 0

            lax.fori_loop(0, n_chunks, chunk_body, 0)
            pltpu.sync_copy(cnt_v, cnt_hbm.at[wid])

    return prologue


def _make_sc_layer(N, E):
    C = 16384
    n_chunks = E // C
    n_pairs = n_chunks // 2
    U = 8

    @functools.partial(
        pl.kernel,
        mesh=_sc_mesh(),
        out_type=jax.ShapeDtypeStruct((_NET, 64, N), jnp.float32),
        compiler_params=pltpu.CompilerParams(needs_layout_passes=False),
        scratch_types=[
            pltpu.VMEM((2, N), jnp.float32),
            pltpu.VMEM((2, N), jnp.float32),
            pltpu.VMEM((C,), jnp.int32),
            pltpu.VMEM((C,), jnp.int32),
            pltpu.SemaphoreType.DMA,
            pltpu.SemaphoreType.DMA,
        ],
    )
    def sc_layer(xs_hbm, sd_hbm, out_hbm, acc_v, y_v,
                 sd_a, sd_b, sem_a, sem_b):
        wid = lax.axis_index("s") * 2 + lax.axis_index("c")
        d0 = wid * 2
        d_idx = [jnp.full((16,), d, jnp.int32) for d in range(2)]
        zeros16 = jnp.zeros((16,), jnp.float32)
        mask16 = jnp.full((16,), 0xFFFF, jnp.int32)

        def issue(i, c, sv, sem):
            pltpu.async_copy(sd_hbm.at[i, pl.ds(c * C, C)], sv, sem)

        def drain(i, sv, sem):
            pltpu.make_async_copy(sd_hbm.at[i, pl.ds(0, C)], sv, sem).wait()

        def process(sv):
            @plsc.parallel_loop(0, C // 16, unroll=U)
            def _(k):
                b = k * 16
                sd = sv[pl.ds(b, 16)]
                si = lax.bitwise_and(sd, mask16)
                di = lax.shift_right_logical(sd, 16)
                for d in range(2):
                    v = plsc.load_gather(y_v, [d_idx[d], si])
                    plsc.addupdate_scatter(acc_v, [d_idx[d], di], v)

        for i in range(_NET):
            s_type = _ET_PAIRS[i][0]
            pltpu.sync_copy(xs_hbm.at[s_type, pl.ds(d0, 2)], y_v)

            @plsc.parallel_loop(0, N // 16, unroll=8)
            def _(k):
                b = k * 16
                for d in range(2):
                    acc_v[d, pl.ds(b, 16)] = zeros16

            issue(i, 0, sd_a, sem_a)

            def pair_body(g, _):
                issue(i, 2 * g + 1, sd_b, sem_b)
                drain(i, sd_a, sem_a)
                process(sd_a)

                @pl.when(g < n_pairs - 1)
                def _():
                    issue(i, 2 * g + 2, sd_a, sem_a)

                drain(i, sd_b, sem_b)
                process(sd_b)
                return 0

            lax.fori_loop(0, n_pairs, pair_body, 0)
            pltpu.sync_copy(acc_v, out_hbm.at[i, pl.ds(d0, 2)])

    return sc_layer


import numpy as _np

_2PI = 2.0 * math.pi
_P1 = float(_np.float32(6.28125))
_P2 = float(_np.float32(_2PI - _P1))
_P3 = float(_np.float32(_2PI - _P1 - float(_np.float32(_2PI - _P1))))


def _dot(a, b):
    return jax.lax.dot_general(
        a.astype(jnp.bfloat16), b.astype(jnp.bfloat16),
        (((1,), (0,)), ((), ())),
        preferred_element_type=jnp.float32)


def _dot_hi(a, b):
    return jax.lax.dot_general(
        a, b, (((1,), (0,)), ((), ())),
        precision=jax.lax.Precision.HIGHEST,
        preferred_element_type=jnp.float32)


def _sincos(x):
    k = jnp.floor(x * (1.0 / _2PI) + 0.5)
    r = ((x - k * _P1) - k * _P2) - k * _P3
    return jnp.sin(r), jnp.cos(r)


def _emb_body(fbox, fpf, fobj, fcyl, times, wemb, bemb, out):
    feats = {0: fbox, 1: fpf, 2: fobj, 3: fcyl}
    bn = times.shape[1]
    for t in range(_NT):
        tt = times[t:t + 1, :].astype(jnp.float32)
        s1, c1 = _sincos(tt)
        s2, c2 = _sincos(tt * 0.01)
        pe = [s1, c1, s2, c2]
        if t <= 2:
            rows = [feats[t][:, :]] + pe
        elif t == 3:
            rows = [feats[3][:, :]] + pe + [jnp.zeros((1, bn), jnp.float32)]
        else:
            rows = pe + [jnp.zeros((4, bn), jnp.float32)]
        inp = jnp.concatenate(rows, axis=0)
        out[t] = _dot(wemb[t], inp) + bemb[t][:, None]


def _make_emb(N):
    BN = 2048
    return pl.pallas_call(
        _emb_body,
        grid=(N // BN,),
        in_specs=[
            pl.BlockSpec((4, BN), lambda nb: (0, nb)),
            pl.BlockSpec((4, BN), lambda nb: (0, nb)),
            pl.BlockSpec((4, BN), lambda nb: (0, nb)),
            pl.BlockSpec((3, BN), lambda nb: (0, nb)),
            pl.BlockSpec((_NT, BN), lambda nb: (0, nb)),
            pl.BlockSpec((_NT, 64, 8), lambda nb: (0, 0, 0)),
            pl.BlockSpec((_NT, 64), lambda nb: (0, 0)),
        ],
        out_specs=pl.BlockSpec((_NT, 64, BN), lambda nb: (0, 0, nb)),
        out_shape=jax.ShapeDtypeStruct((_NT, 64, N), jnp.float32),
    )


def _layer_body(s, c, xs, wl, wr, bias, out):
    for t in range(_NT):
        o = None
        for i in _DST_GROUP[t]:
            cc = jnp.maximum(c[i:i + 1, :], 1.0)
            agg = s[i] / cc
            m = _dot(wl[i], agg) + bias[i][:, None]
            m = m + _dot(wr[i], xs[t])
            o = m if o is None else o + m
        out[t] = jnp.maximum(o, 0.0)


def _make_layer_tc(N):
    BN = 1024
    return pl.pallas_call(
        _layer_body,
        grid=(N // BN,),
        in_specs=[
            pl.BlockSpec((_NET, 64, BN), lambda nb: (0, 0, nb)),
            pl.BlockSpec((_NET, BN), lambda nb: (0, nb)),
            pl.BlockSpec((_NT, 64, BN), lambda nb: (0, 0, nb)),
            pl.BlockSpec((_NET, 64, 64), lambda nb: (0, 0, 0)),
            pl.BlockSpec((_NET, 64, 64), lambda nb: (0, 0, 0)),
            pl.BlockSpec((_NET, 64), lambda nb: (0, 0)),
        ],
        out_specs=pl.BlockSpec((_NT, 64, BN), lambda nb: (0, 0, nb)),
        out_shape=jax.ShapeDtypeStruct((_NT, 64, N), jnp.float32),
    )


def _head_body(x, w1, b1, w2, b2, out):
    h = jnp.maximum(x[0], 0.0)
    o1 = _dot(w1[...], h) + b1[0][:, None]
    o1 = jnp.maximum(o1, 0.0)
    o2 = _dot(w2[...], o1) + b2[0][:, None]
    out[0] = o2


def _make_head(N, HQ):
    BN = 2048
    return pl.pallas_call(
        _head_body,
        grid=(2, N // BN),
        in_specs=[
            pl.BlockSpec((1, 64, BN), lambda j, nb: (4 + j, 0, nb)),
            pl.BlockSpec((HQ, 64), lambda j, nb: (0, 0)),
            pl.BlockSpec((1, HQ), lambda j, nb: (0, 0)),
            pl.BlockSpec((1, HQ), lambda j, nb: (0, 0)),
            pl.BlockSpec((1, 1), lambda j, nb: (0, 0)),
        ],
        out_specs=pl.BlockSpec((1, 1, BN), lambda j, nb: (j, 0, nb)),
        out_shape=jax.ShapeDtypeStruct((2, 1, N), jnp.float32),
    )


def kernel(x_object, x_ssBox, x_place_frame, x_ssCylinder, times_all,
           actives_all, edge_all, num_pick_nodes, num_place_nodes,
           W_emb_ssBox, b_emb_ssBox, W_emb_place_frame, b_emb_place_frame,
           W_emb_object, b_emb_object, W_emb_ssCylinder, b_emb_ssCylinder,
           W_emb_pick, b_emb_pick, W_emb_place, b_emb_place,
           Wl, bl, Wr, W_out1, b_out1, W_out2, b_out2):
    N = x_object.shape[0]
    E = edge_all.shape[2]
    L = Wl.shape[0]

    times = jnp.minimum(times_all, 2 * num_pick_nodes - 1).astype(jnp.int32)
    src24 = edge_all[:, 0, :].astype(jnp.int32)
    dst24 = edge_all[:, 1, :].astype(jnp.int32)

    fbox = x_ssBox.T
    fpf = x_place_frame.T
    fobj = x_object.T
    fcyl = x_ssCylinder.T

    pad_c = lambda w, k: jnp.pad(w, ((0, 0), (0, k)))
    wemb = jnp.stack([
        W_emb_ssBox, W_emb_place_frame, W_emb_object,
        pad_c(W_emb_ssCylinder, 1), pad_c(W_emb_pick, 4), pad_c(W_emb_place, 4),
    ])
    bemb = jnp.stack([
        b_emb_ssBox, b_emb_place_frame, b_emb_object,
        b_emb_ssCylinder, b_emb_pick, b_emb_place,
    ])

    cnt = _make_prologue(N, E)(dst24)
    xs = _make_emb(N)(fbox, fpf, fobj, fcyl, times, wemb, bemb)

    sd24 = jnp.bitwise_or(src24, jnp.left_shift(dst24, 16))

    layer_tc = _make_layer_tc(N)
    sc_layer = _make_sc_layer(N, E)
    for l in range(L):
        s = sc_layer(xs, sd24)
        xs = layer_tc(s, cnt, xs, Wl[l], Wr[l], bl[l])

    hq = W_out1.shape[0]
    out = _make_head(N, hq)(xs, W_out1, b_out1[None, :], W_out2,
                            b_out2[None, :])
    return out.reshape(2 * N, 1)


# speedup vs baseline: 7.2927x; 1.0771x over previous
import functools
import math

import jax
import jax.numpy as jnp
from jax import lax
from jax.experimental import pallas as pl
from jax.experimental.pallas import tpu as pltpu
from jax.experimental.pallas import tpu_sc as plsc

_ET_PAIRS = [
    (2, 0), (0, 2), (1, 0), (0, 1), (1, 2), (2, 1),
    (4, 5), (5, 4), (2, 2), (0, 0), (1, 1), (3, 3),
    (2, 4), (4, 2), (1, 4), (4, 1), (3, 4), (4, 3),
    (2, 5), (5, 2), (3, 5), (5, 3), (1, 5), (5, 1),
]
_NT = 6
_NET = len(_ET_PAIRS)

_DST_GROUP = [[i for i, (s, d) in enumerate(_ET_PAIRS) if d == t] for t in range(_NT)]


def _sc_mesh():
    return plsc.VectorSubcoreMesh(core_axis_name="c", subcore_axis_name="s")


def _make_prologue(N, E):
    C = 16384
    n_chunks = E // C

    @functools.partial(
        pl.kernel,
        mesh=_sc_mesh(),
        out_type=jax.ShapeDtypeStruct((_NET, N), jnp.float32),
        compiler_params=pltpu.CompilerParams(needs_layout_passes=False),
        scratch_types=[
            pltpu.VMEM((N,), jnp.float32),
            pltpu.VMEM((C,), jnp.int32),
        ],
    )
    def prologue(dst_hbm, cnt_hbm, cnt_v, di_v):
        wid = lax.axis_index("s") * 2 + lax.axis_index("c")

        @pl.when(wid < _NET)
        def _():
            zeros16 = jnp.zeros((16,), jnp.float32)
            ones16 = jnp.ones((16,), jnp.float32)

            @plsc.parallel_loop(0, N // 16, unroll=8)
            def _(k):
                cnt_v[pl.ds(k * 16, 16)] = zeros16

            def chunk_body(c, _):
                pltpu.sync_copy(dst_hbm.at[wid, pl.ds(c * C, C)], di_v)

                @plsc.parallel_loop(0, C // 16, unroll=8)
                def _(k):
                    di = di_v[pl.ds(k * 16, 16)]
                    plsc.addupdate_scatter(cnt_v, [di], ones16)

                return 0

            lax.fori_loop(0, n_chunks, chunk_body, 0)
            pltpu.sync_copy(cnt_v, cnt_hbm.at[wid])

    return prologue


def _make_sc_layer(N, E):
    C = 16384
    n_chunks = E // C
    n_pairs = n_chunks // 2
    U = 8

    @functools.partial(
        pl.kernel,
        mesh=_sc_mesh(),
        out_type=jax.ShapeDtypeStruct((_NET, 64, N), jnp.float32),
        compiler_params=pltpu.CompilerParams(needs_layout_passes=False),
        scratch_types=[
            pltpu.VMEM((2, N), jnp.float32),
            pltpu.VMEM((2, N), jnp.float32),
            pltpu.VMEM((C,), jnp.int32),
            pltpu.VMEM((C,), jnp.int32),
            pltpu.SemaphoreType.DMA,
            pltpu.SemaphoreType.DMA,
            pltpu.SemaphoreType.DMA,
        ],
    )
    def sc_layer(xs_hbm, sd_hbm, out_hbm, acc_v, y_v,
                 sd_a, sd_b, sem_a, sem_b, sem_y):
        wid = lax.axis_index("s") * 2 + lax.axis_index("c")
        d0 = wid * 2
        d_idx = [jnp.full((16,), d, jnp.int32) for d in range(2)]
        zeros16 = jnp.zeros((16,), jnp.float32)
        mask16 = jnp.full((16,), 0xFFFF, jnp.int32)

        def issue(i, c, sv, sem):
            pltpu.async_copy(sd_hbm.at[i, pl.ds(c * C, C)], sv, sem)

        def drain(i, sv, sem):
            pltpu.make_async_copy(sd_hbm.at[i, pl.ds(0, C)], sv, sem).wait()

        def process(sv):
            @plsc.parallel_loop(0, C // 16, unroll=U)
            def _(k):
                b = k * 16
                sd = sv[pl.ds(b, 16)]
                si = lax.bitwise_and(sd, mask16)
                di = lax.shift_right_logical(sd, 16)
                for d in range(2):
                    v = plsc.load_gather(y_v, [d_idx[d], si])
                    plsc.addupdate_scatter(acc_v, [d_idx[d], di], v)

        for i in range(_NET):
            s_type = _ET_PAIRS[i][0]
            issue(i, 0, sd_a, sem_a)
            y_cp = pltpu.async_copy(xs_hbm.at[s_type, pl.ds(d0, 2)], y_v,
                                    sem_y)

            @plsc.parallel_loop(0, N // 16, unroll=8)
            def _(k):
                b = k * 16
                for d in range(2):
                    acc_v[d, pl.ds(b, 16)] = zeros16

            y_cp.wait()

            def pair_body(g, _):
                issue(i, 2 * g + 1, sd_b, sem_b)
                drain(i, sd_a, sem_a)
                process(sd_a)

                @pl.when(g < n_pairs - 1)
                def _():
                    issue(i, 2 * g + 2, sd_a, sem_a)

                drain(i, sd_b, sem_b)
                process(sd_b)
                return 0

            lax.fori_loop(0, n_pairs, pair_body, 0)
            pltpu.sync_copy(acc_v, out_hbm.at[i, pl.ds(d0, 2)])

    return sc_layer


import numpy as _np

_2PI = 2.0 * math.pi
_P1 = float(_np.float32(6.28125))
_P2 = float(_np.float32(_2PI - _P1))
_P3 = float(_np.float32(_2PI - _P1 - float(_np.float32(_2PI - _P1))))


def _dot(a, b):
    return jax.lax.dot_general(
        a.astype(jnp.bfloat16), b.astype(jnp.bfloat16),
        (((1,), (0,)), ((), ())),
        preferred_element_type=jnp.float32)


def _dot_hi(a, b):
    return jax.lax.dot_general(
        a, b, (((1,), (0,)), ((), ())),
        precision=jax.lax.Precision.HIGHEST,
        preferred_element_type=jnp.float32)


def _sincos(x):
    k = jnp.floor(x * (1.0 / _2PI) + 0.5)
    r = ((x - k * _P1) - k * _P2) - k * _P3
    return jnp.sin(r), jnp.cos(r)


def _emb_body(fbox, fpf, fobj, fcyl, times, wemb, bemb, out):
    feats = {0: fbox, 1: fpf, 2: fobj, 3: fcyl}
    bn = times.shape[1]
    for t in range(_NT):
        tt = times[t:t + 1, :].astype(jnp.float32)
        s1, c1 = _sincos(tt)
        s2, c2 = _sincos(tt * 0.01)
        pe = [s1, c1, s2, c2]
        if t <= 2:
            rows = [feats[t][:, :]] + pe
        elif t == 3:
            rows = [feats[3][:, :]] + pe + [jnp.zeros((1, bn), jnp.float32)]
        else:
            rows = pe + [jnp.zeros((4, bn), jnp.float32)]
        inp = jnp.concatenate(rows, axis=0)
        out[t] = _dot(wemb[t], inp) + bemb[t][:, None]


def _make_emb(N):
    BN = 2048
    return pl.pallas_call(
        _emb_body,
        grid=(N // BN,),
        in_specs=[
            pl.BlockSpec((4, BN), lambda nb: (0, nb)),
            pl.BlockSpec((4, BN), lambda nb: (0, nb)),
            pl.BlockSpec((4, BN), lambda nb: (0, nb)),
            pl.BlockSpec((3, BN), lambda nb: (0, nb)),
            pl.BlockSpec((_NT, BN), lambda nb: (0, nb)),
            pl.BlockSpec((_NT, 64, 8), lambda nb: (0, 0, 0)),
            pl.BlockSpec((_NT, 64), lambda nb: (0, 0)),
        ],
        out_specs=pl.BlockSpec((_NT, 64, BN), lambda nb: (0, 0, nb)),
        out_shape=jax.ShapeDtypeStruct((_NT, 64, N), jnp.float32),
    )


def _layer_body(s, c, xs, wl, wr, bias, out):
    for t in range(_NT):
        o = None
        for i in _DST_GROUP[t]:
            cc = jnp.maximum(c[i:i + 1, :], 1.0)
            agg = s[i] / cc
            m = _dot(wl[i], agg) + bias[i][:, None]
            m = m + _dot(wr[i], xs[t])
            o = m if o is None else o + m
        out[t] = jnp.maximum(o, 0.0)


def _make_layer_tc(N):
    BN = 1024
    return pl.pallas_call(
        _layer_body,
        grid=(N // BN,),
        in_specs=[
            pl.BlockSpec((_NET, 64, BN), lambda nb: (0, 0, nb)),
            pl.BlockSpec((_NET, BN), lambda nb: (0, nb)),
            pl.BlockSpec((_NT, 64, BN), lambda nb: (0, 0, nb)),
            pl.BlockSpec((_NET, 64, 64), lambda nb: (0, 0, 0)),
            pl.BlockSpec((_NET, 64, 64), lambda nb: (0, 0, 0)),
            pl.BlockSpec((_NET, 64), lambda nb: (0, 0)),
        ],
        out_specs=pl.BlockSpec((_NT, 64, BN), lambda nb: (0, 0, nb)),
        out_shape=jax.ShapeDtypeStruct((_NT, 64, N), jnp.float32),
    )


def _head_body(x, w1, b1, w2, b2, out):
    h = jnp.maximum(x[0], 0.0)
    o1 = _dot(w1[...], h) + b1[0][:, None]
    o1 = jnp.maximum(o1, 0.0)
    o2 = _dot(w2[...], o1) + b2[0][:, None]
    out[0] = o2


def _make_head(N, HQ):
    BN = 2048
    return pl.pallas_call(
        _head_body,
        grid=(2, N // BN),
        in_specs=[
            pl.BlockSpec((1, 64, BN), lambda j, nb: (4 + j, 0, nb)),
            pl.BlockSpec((HQ, 64), lambda j, nb: (0, 0)),
            pl.BlockSpec((1, HQ), lambda j, nb: (0, 0)),
            pl.BlockSpec((1, HQ), lambda j, nb: (0, 0)),
            pl.BlockSpec((1, 1), lambda j, nb: (0, 0)),
        ],
        out_specs=pl.BlockSpec((1, 1, BN), lambda j, nb: (j, 0, nb)),
        out_shape=jax.ShapeDtypeStruct((2, 1, N), jnp.float32),
    )


def kernel(x_object, x_ssBox, x_place_frame, x_ssCylinder, times_all,
           actives_all, edge_all, num_pick_nodes, num_place_nodes,
           W_emb_ssBox, b_emb_ssBox, W_emb_place_frame, b_emb_place_frame,
           W_emb_object, b_emb_object, W_emb_ssCylinder, b_emb_ssCylinder,
           W_emb_pick, b_emb_pick, W_emb_place, b_emb_place,
           Wl, bl, Wr, W_out1, b_out1, W_out2, b_out2):
    N = x_object.shape[0]
    E = edge_all.shape[2]
    L = Wl.shape[0]

    times = jnp.minimum(times_all, 2 * num_pick_nodes - 1).astype(jnp.int32)
    src24 = edge_all[:, 0, :].astype(jnp.int32)
    dst24 = edge_all[:, 1, :].astype(jnp.int32)

    fbox = x_ssBox.T
    fpf = x_place_frame.T
    fobj = x_object.T
    fcyl = x_ssCylinder.T

    pad_c = lambda w, k: jnp.pad(w, ((0, 0), (0, k)))
    wemb = jnp.stack([
        W_emb_ssBox, W_emb_place_frame, W_emb_object,
        pad_c(W_emb_ssCylinder, 1), pad_c(W_emb_pick, 4), pad_c(W_emb_place, 4),
    ])
    bemb = jnp.stack([
        b_emb_ssBox, b_emb_place_frame, b_emb_object,
        b_emb_ssCylinder, b_emb_pick, b_emb_place,
    ])

    cnt = _make_prologue(N, E)(dst24)
    xs = _make_emb(N)(fbox, fpf, fobj, fcyl, times, wemb, bemb)

    sd24 = jnp.bitwise_or(src24, jnp.left_shift(dst24, 16))

    layer_tc = _make_layer_tc(N)
    sc_layer = _make_sc_layer(N, E)
    for l in range(L):
        s = sc_layer(xs, sd24)
        xs = layer_tc(s, cnt, xs, Wl[l], Wr[l], bl[l])

    hq = W_out1.shape[0]
    out = _make_head(N, hq)(xs, W_out1, b_out1[None, :], W_out2,
                            b_out2[None, :])
    return out.reshape(2 * N, 1)
